# Initial kernel scaffold; baseline (speedup 1.0000x reference)
import jax
import jax.numpy as jnp
from jax.experimental import pallas as pl


def kernel(input_ids, wte):
    raise NotImplementedError("write your pallas kernel here")


# speedup vs baseline: 1.8765x; 1.8765x over previous
import functools

import jax
import jax.numpy as jnp
from jax import lax
from jax.experimental import pallas as pl
from jax.experimental.pallas import tpu as pltpu
from jax.experimental.pallas import tpu_sc as plsc

NUM_WORKERS = 32
EMBED = 64
TOTAL_ROWS = 16384 * 50
ROWS_PER_WORKER = TOTAL_ROWS // NUM_WORKERS
CHUNK = 512
NUM_CHUNKS = ROWS_PER_WORKER // CHUNK


def _body(idx_hbm, wte_hbm, out_hbm, idx_v, rows0, rows1, sem0, sem1):
    wid = lax.axis_index("s") * 2 + lax.axis_index("c")
    base = wid * ROWS_PER_WORKER
    pltpu.sync_copy(idx_hbm.at[pl.ds(base, ROWS_PER_WORKER)], idx_v)

    def gather(chunk, buf, sem):
        off = pl.multiple_of(chunk * CHUNK, CHUNK)
        return pltpu.async_copy(wte_hbm.at[idx_v.at[pl.ds(off, CHUNK)]], buf, sem)

    def store(chunk, buf):
        off = pl.multiple_of(base + chunk * CHUNK, CHUNK)
        pltpu.sync_copy(buf, out_hbm.at[pl.ds(off, CHUNK)])

    gather(0, rows0, sem0)
    gather(1, rows1, sem1)

    def step(g, _):
        c = pl.multiple_of(g * 2, 2)
        pltpu.make_async_copy(wte_hbm.at[idx_v.at[pl.ds(0, CHUNK)]], rows0, sem0).wait()
        store(c, rows0)
        gather(c + 2, rows0, sem0)
        pltpu.make_async_copy(wte_hbm.at[idx_v.at[pl.ds(0, CHUNK)]], rows1, sem1).wait()
        store(c + 1, rows1)
        gather(c + 3, rows1, sem1)
        return 0

    lax.fori_loop(0, NUM_CHUNKS // 2 - 1, step, 0)

    pltpu.make_async_copy(wte_hbm.at[idx_v.at[pl.ds(0, CHUNK)]], rows0, sem0).wait()
    store(NUM_CHUNKS - 2, rows0)
    pltpu.make_async_copy(wte_hbm.at[idx_v.at[pl.ds(0, CHUNK)]], rows1, sem1).wait()
    store(NUM_CHUNKS - 1, rows1)


@jax.jit
def _lookup(idx_flat, wte):
    mesh = plsc.VectorSubcoreMesh(core_axis_name="c", subcore_axis_name="s")
    return pl.kernel(
        _body,
        out_type=jax.ShapeDtypeStruct((TOTAL_ROWS, EMBED), jnp.float32),
        mesh=mesh,
        scratch_types=[
            pltpu.VMEM((ROWS_PER_WORKER,), jnp.int32),
            pltpu.VMEM((CHUNK, EMBED), jnp.float32),
            pltpu.VMEM((CHUNK, EMBED), jnp.float32),
            pltpu.SemaphoreType.DMA,
            pltpu.SemaphoreType.DMA,
        ],
        compiler_params=pltpu.CompilerParams(use_tc_tiling_on_sc=False),
    )(idx_flat, wte)


def kernel(input_ids, wte):
    b, t = input_ids.shape
    out = _lookup(input_ids.reshape(-1).astype(jnp.int32), wte)
    return out.reshape(b, t, EMBED)


# speedup vs baseline: 2.4284x; 1.2941x over previous
import functools

import jax
import jax.numpy as jnp
from jax import lax
from jax.experimental import pallas as pl
from jax.experimental.pallas import tpu as pltpu
from jax.experimental.pallas import tpu_sc as plsc

NUM_WORKERS = 32
EMBED = 64
TOK = 50
BATCH = 16384
BW = BATCH // NUM_WORKERS


def _transpose_chunk(buf, tbuf):
    lanes = lax.iota(jnp.int32, 16)
    rowsk = [k * 16 + lanes for k in range(EMBED // 16)]

    @plsc.parallel_loop(0, BW, 1, unroll=4)
    def j_body(j):
        col = jnp.full((16,), j, jnp.int32)
        for k in range(EMBED // 16):
            vals = buf[j, pl.ds(k * 16, 16)]
            plsc.store_scatter(tbuf, [rowsk[k], col], vals)


def _body(ids_hbm, wte_hbm, out_hbm, idx_v, rows0, rows1, tbuf, sem0, sem1):
    wid = lax.axis_index("s") * 2 + lax.axis_index("c")
    b0 = pl.multiple_of(wid * BW, BW)
    pltpu.sync_copy(ids_hbm.at[:, pl.ds(b0, BW)], idx_v)

    def gather(t, buf, sem):
        return pltpu.async_copy(wte_hbm.at[idx_v.at[t]], buf, sem)

    def wait(buf, sem):
        pltpu.make_async_copy(wte_hbm.at[idx_v.at[0]], buf, sem).wait()

    def put(t, src):
        pltpu.sync_copy(
            src.at[:, pl.ds(0, BW)],
            out_hbm.at[pl.ds(t * EMBED, EMBED), pl.ds(b0, BW)],
        )

    gather(0, rows0, sem0)
    gather(1, rows1, sem1)

    def step(i, _):
        t = pl.multiple_of(i * 2, 2)
        wait(rows0, sem0)
        _transpose_chunk(rows0, tbuf)
        gather(t + 2, rows0, sem0)
        put(t, tbuf)
        wait(rows1, sem1)
        _transpose_chunk(rows1, tbuf)
        gather(t + 3, rows1, sem1)
        put(t + 1, tbuf)
        return 0

    lax.fori_loop(0, TOK // 2 - 1, step, 0)

    wait(rows0, sem0)
    _transpose_chunk(rows0, tbuf)
    put(TOK - 2, tbuf)
    wait(rows1, sem1)
    _transpose_chunk(rows1, tbuf)
    put(TOK - 1, tbuf)


@jax.jit
def _lookup(ids2d, wte):
    mesh = plsc.VectorSubcoreMesh(core_axis_name="c", subcore_axis_name="s")
    return pl.kernel(
        _body,
        out_type=jax.ShapeDtypeStruct((TOK * EMBED, BATCH), jnp.float32),
        mesh=mesh,
        scratch_types=[
            pltpu.VMEM((TOK, BW), jnp.int32),
            pltpu.VMEM((BW, EMBED), jnp.float32),
            pltpu.VMEM((BW, EMBED), jnp.float32),
            pltpu.VMEM((EMBED, BW + 1), jnp.float32),
            pltpu.SemaphoreType.DMA,
            pltpu.SemaphoreType.DMA,
        ],
        compiler_params=pltpu.CompilerParams(
            use_tc_tiling_on_sc=False, needs_layout_passes=False
        ),
    )(ids2d, wte)


def kernel(input_ids, wte):
    ids2d = input_ids.T.astype(jnp.int32)
    out = _lookup(ids2d, wte)
    return out.reshape(TOK, EMBED, BATCH).transpose(2, 0, 1)


# speedup vs baseline: 2.4312x; 1.0011x over previous
import functools

import jax
import jax.numpy as jnp
from jax import lax
from jax.experimental import pallas as pl
from jax.experimental.pallas import tpu as pltpu
from jax.experimental.pallas import tpu_sc as plsc

NUM_WORKERS = 32
EMBED = 64
TOK = 50
BATCH = 16384
BW = BATCH // NUM_WORKERS
HB = BW // 2


def _transpose_half(buf, tbuf, r0):
    lanes = lax.iota(jnp.int32, 16)
    rowsk = [k * 16 + lanes for k in range(EMBED // 16)]

    @plsc.parallel_loop(r0, r0 + HB, 1, unroll=4)
    def j_body(j):
        col = jnp.full((16,), j, jnp.int32)
        for k in range(EMBED // 16):
            vals = buf[j, pl.ds(k * 16, 16)]
            plsc.store_scatter(tbuf, [rowsk[k], col], vals)


def _body(
    ids_hbm,
    wte_hbm,
    out_hbm,
    idx_v,
    rows0,
    rows1,
    tbuf,
    sem0,
    sem1,
    osem0,
    osem1,
):
    wid = lax.axis_index("s") * 2 + lax.axis_index("c")
    b0 = pl.multiple_of(wid * BW, BW)
    pltpu.sync_copy(ids_hbm.at[:, pl.ds(b0, BW)], idx_v)

    def gather(t, buf, sem):
        return pltpu.async_copy(wte_hbm.at[idx_v.at[t]], buf, sem)

    def wait(buf, sem):
        pltpu.make_async_copy(wte_hbm.at[idx_v.at[0]], buf, sem).wait()

    def put_half(t, h, osem):
        pltpu.async_copy(
            tbuf.at[:, pl.ds(h * HB, HB)],
            out_hbm.at[t, :, pl.ds(b0 + h * HB, HB)],
            osem,
        )

    def put_half_wait(h, osem):
        pltpu.make_async_copy(
            tbuf.at[:, pl.ds(h * HB, HB)],
            out_hbm.at[0, :, pl.ds(b0 + h * HB, HB)],
            osem,
        ).wait()

    def proc(buf, sem, t, issue_next):
        wait(buf, sem)
        put_half_wait(0, osem0)
        _transpose_half(buf, tbuf, 0)
        put_half(t, 0, osem0)
        put_half_wait(1, osem1)
        _transpose_half(buf, tbuf, HB)
        if issue_next:
            gather(t + 2, buf, sem)
        put_half(t, 1, osem1)

    gather(0, rows0, sem0)
    gather(1, rows1, sem1)

    wait(rows0, sem0)
    _transpose_half(rows0, tbuf, 0)
    put_half(0, 0, osem0)
    _transpose_half(rows0, tbuf, HB)
    gather(2, rows0, sem0)
    put_half(0, 1, osem1)

    def step(i, _):
        t1 = pl.multiple_of(i * 2, 2) + 1
        proc(rows1, sem1, t1, True)
        proc(rows0, sem0, t1 + 1, True)
        return 0

    lax.fori_loop(0, (TOK - 4) // 2, step, 0)

    proc(rows1, sem1, TOK - 3, True)
    proc(rows0, sem0, TOK - 2, False)
    proc(rows1, sem1, TOK - 1, False)
    put_half_wait(0, osem0)
    put_half_wait(1, osem1)


@jax.jit
def _lookup(ids2d, wte):
    mesh = plsc.VectorSubcoreMesh(core_axis_name="c", subcore_axis_name="s")
    return pl.kernel(
        _body,
        out_type=jax.ShapeDtypeStruct((TOK, EMBED, BATCH), jnp.float32),
        mesh=mesh,
        scratch_types=[
            pltpu.VMEM((TOK, BW), jnp.int32),
            pltpu.VMEM((BW, EMBED), jnp.float32),
            pltpu.VMEM((BW, EMBED), jnp.float32),
            pltpu.VMEM((EMBED, BW + 1), jnp.float32),
            pltpu.SemaphoreType.DMA,
            pltpu.SemaphoreType.DMA,
            pltpu.SemaphoreType.DMA,
            pltpu.SemaphoreType.DMA,
        ],
        compiler_params=pltpu.CompilerParams(
            use_tc_tiling_on_sc=False, needs_layout_passes=False
        ),
    )(ids2d, wte)


def kernel(input_ids, wte):
    ids2d = input_ids.T.astype(jnp.int32)
    out = _lookup(ids2d, wte)
    return out.transpose(2, 0, 1)
